# Initial kernel scaffold; baseline (speedup 1.0000x reference)
import jax
import jax.numpy as jnp
from jax.experimental import pallas as pl


def kernel(pos, batch):
    raise NotImplementedError("write your pallas kernel here")


# speedup vs baseline: 115.6344x; 115.6344x over previous
import functools

import jax
import jax.numpy as jnp
from jax import lax
from jax.experimental import pallas as pl
from jax.experimental.pallas import tpu as pltpu
from jax.experimental.pallas import tpu_sc as plsc

N = 10000
K = 32
CUTOFF2 = 100.0
NGRAPH = 64
NPAD = 10016
NBLK = NPAD // 16
NC, NS, L = 2, 16, 16
NW = NC * NS
SPAD = 80
ROWS2D = (NPAD * K) // 128


def _lane_max_i32(vec, mask):
    return jnp.max(jnp.where(mask, vec, jnp.int32(-2147483648)))


def _lane_max_f32(vec, mask):
    return jnp.max(jnp.where(mask, vec, jnp.float32(-jnp.inf)))


def _merge_topk(W0k, W0v, W1k, W1v, sk, sv):
    sk, sv = plsc.sort_key_val(sk, sv)
    rsk = jnp.flip(sk, 0)
    rsv = jnp.flip(sv, 0)
    m = W1k <= rsk
    lk = jnp.where(m, W1k, rsk)
    lv = jnp.where(m, W1v, rsv)
    lk, lv = plsc.sort_key_val(lk, lv)
    rlk = jnp.flip(lk, 0)
    rlv = jnp.flip(lv, 0)
    m2 = W0k <= rlk
    mk = jnp.where(m2, W0k, rlk)
    mv = jnp.where(m2, W0v, rlv)
    xk = jnp.where(m2, rlk, W0k)
    xv = jnp.where(m2, rlv, W0v)
    W0k, W0v = plsc.sort_key_val(mk, mv)
    W1k, W1v = plsc.sort_key_val(xk, xv)
    return W0k, W0v, W1k, W1v


@functools.partial(
    pl.kernel,
    out_type=(
        jax.ShapeDtypeStruct((NPAD * K,), jnp.int32),
        jax.ShapeDtypeStruct((NPAD * K,), jnp.float32),
    ),
    mesh=plsc.VectorSubcoreMesh(
        core_axis_name="c", subcore_axis_name="s", num_cores=NC, num_subcores=NS
    ),
    compiler_params=pltpu.CompilerParams(needs_layout_passes=False),
    scratch_types=[
        pltpu.VMEM((NPAD,), jnp.float32),
        pltpu.VMEM((NPAD,), jnp.float32),
        pltpu.VMEM((NPAD,), jnp.float32),
        pltpu.VMEM((NPAD,), jnp.int32),
        pltpu.VMEM((SPAD,), jnp.int32),
        pltpu.VMEM((16 * K,), jnp.int32),
        pltpu.VMEM((16 * K,), jnp.float32),
    ],
)
def _sc_topk(xh, yh, zh, bh, sh, col_h, d2_h, xv, yv, zv, bv, sv, colb, d2b):
    wid = lax.axis_index("s") * NC + lax.axis_index("c")
    pltpu.sync_copy(xh, xv)
    pltpu.sync_copy(yh, yv)
    pltpu.sync_copy(zh, zv)
    pltpu.sync_copy(bh, bv)
    pltpu.sync_copy(sh, sv)

    lo = (wid * NBLK) // NW
    hi = ((wid + 1) * NBLK) // NW
    lane = lax.iota(jnp.int32, 16)
    inf = jnp.float32(jnp.inf)
    intmin = jnp.int32(-2147483648)

    stab = [sv[pl.ds(i * 16, 16)] for i in range(SPAD // 16)]

    def starts_at(g):
        best = jnp.full((16,), intmin)
        for i, vec in enumerate(stab):
            best = jnp.maximum(best, jnp.where(lane + i * 16 == g, vec, intmin))
        return jnp.max(best)

    def qblock(qb, carry):
        base = qb * 16
        qxv = xv[pl.ds(base, 16)]
        qyv = yv[pl.ds(base, 16)]
        qzv = zv[pl.ds(base, 16)]
        gvec = bv[pl.ds(base, 16)]

        for l in range(16):
            qidx = base + l

            @pl.when(qidx < N)
            def _():
                lm = lane == l
                g = _lane_max_i32(gvec, lm)
                s = starts_at(g)
                e = starts_at(g + 1)
                qx = jnp.full((16,), _lane_max_f32(qxv, lm))
                qy = jnp.full((16,), _lane_max_f32(qyv, lm))
                qz = jnp.full((16,), _lane_max_f32(qzv, lm))
                cb0 = s >> 4
                cb1 = (e + 15) >> 4

                def cand(cb, W):
                    W0k, W0v, W1k, W1v = W
                    cbase = cb * 16
                    cx = xv[pl.ds(cbase, 16)]
                    cy = yv[pl.ds(cbase, 16)]
                    cz = zv[pl.ds(cbase, 16)]
                    cidx = cbase + lane
                    dx = qx - cx
                    dy = qy - cy
                    dz = qz - cz
                    d2 = (dx * dx + dy * dy) + dz * dz
                    bad = (
                        (cidx < s)
                        | (cidx >= e)
                        | (cidx == qidx)
                        | (d2 > jnp.float32(CUTOFF2))
                    )
                    skey = jnp.where(bad, inf, d2)
                    return _merge_topk(W0k, W0v, W1k, W1v, skey, cidx)

                W0k = jnp.full((16,), inf)
                W1k = jnp.full((16,), inf)
                W0v = jnp.full((16,), qidx)
                W1v = jnp.full((16,), qidx)
                W0k, W0v, W1k, W1v = lax.fori_loop(
                    cb0, cb1, cand, (W0k, W0v, W1k, W1v)
                )
                pad0 = W0k == inf
                pad1 = W1k == inf
                colb[pl.ds(l * K, 16)] = jnp.where(pad0, qidx, W0v)
                colb[pl.ds(l * K + 16, 16)] = jnp.where(pad1, qidx, W1v)
                d2b[pl.ds(l * K, 16)] = jnp.where(pad0, jnp.float32(0.0), W0k)
                d2b[pl.ds(l * K + 16, 16)] = jnp.where(pad1, jnp.float32(0.0), W1k)

        pltpu.sync_copy(colb, col_h.at[pl.ds(qb * (16 * K), 16 * K)])
        pltpu.sync_copy(d2b, d2_h.at[pl.ds(qb * (16 * K), 16 * K)])
        return carry

    lax.fori_loop(lo, hi, qblock, 0)


def _fin_body(d2_ref, row_ref, w_ref):
    d2 = d2_ref[...]
    lin = (
        lax.broadcasted_iota(jnp.int32, (ROWS2D, 128), 0) * 128
        + lax.broadcasted_iota(jnp.int32, (ROWS2D, 128), 1)
    )
    row_ref[...] = lin >> 5
    w_ref[...] = jnp.where(d2 > 0, jnp.sqrt(d2), jnp.float32(0.0))


_finalize = pl.pallas_call(
    _fin_body,
    out_shape=(
        jax.ShapeDtypeStruct((ROWS2D, 128), jnp.int32),
        jax.ShapeDtypeStruct((ROWS2D, 128), jnp.float32),
    ),
)


def kernel(pos, batch):
    pos = pos.astype(jnp.float32)
    b = batch.astype(jnp.int32)
    x = jnp.pad(pos[:, 0], (0, NPAD - N))
    y = jnp.pad(pos[:, 1], (0, NPAD - N))
    z = jnp.pad(pos[:, 2], (0, NPAD - N))
    bpad = jnp.pad(b, (0, NPAD - N))
    starts = jnp.searchsorted(
        b, jnp.arange(NGRAPH + 1, dtype=jnp.int32)
    ).astype(jnp.int32)
    starts = jnp.pad(starts, (0, SPAD - (NGRAPH + 1)), constant_values=N)

    col, d2 = _sc_topk(x, y, z, bpad, starts)
    row2d, w2d = _finalize(d2.reshape(ROWS2D, 128))

    row = row2d.reshape(-1)[: N * K]
    edge_index = jnp.stack([row, col[: N * K]], axis=0)
    return edge_index, w2d.reshape(-1)[: N * K]


# speedup vs baseline: 217.9898x; 1.8852x over previous
import functools

import jax
import jax.numpy as jnp
from jax import lax
from jax.experimental import pallas as pl
from jax.experimental.pallas import tpu as pltpu
from jax.experimental.pallas import tpu_sc as plsc

N = 10000
K = 32
CUTOFF2 = 100.0
NGRAPH = 64
NBLK = N // 16
NC, NS, L = 2, 16, 16
NW = NC * NS
SPAD = 80
ROWS2D = (N * K) // 128
GQ = 4


def _merge_topk(W0k, W0v, W1k, W1v, sk, sv):
    sk, sv = plsc.sort_key_val(sk, sv)
    rsk = jnp.flip(sk, 0)
    rsv = jnp.flip(sv, 0)
    m = W1k <= rsk
    lk = jnp.where(m, W1k, rsk)
    lv = jnp.where(m, W1v, rsv)
    lk, lv = plsc.sort_key_val(lk, lv)
    rlk = jnp.flip(lk, 0)
    rlv = jnp.flip(lv, 0)
    m2 = W0k <= rlk
    mk = jnp.where(m2, W0k, rlk)
    mv = jnp.where(m2, W0v, rlv)
    xk = jnp.where(m2, rlk, W0k)
    xv = jnp.where(m2, rlv, W0v)
    W0k, W0v = plsc.sort_key_val(mk, mv)
    W1k, W1v = plsc.sort_key_val(xk, xv)
    return W0k, W0v, W1k, W1v


@functools.partial(
    pl.kernel,
    out_type=(
        jax.ShapeDtypeStruct((N * K,), jnp.int32),
        jax.ShapeDtypeStruct((N * K,), jnp.float32),
    ),
    mesh=plsc.VectorSubcoreMesh(
        core_axis_name="c", subcore_axis_name="s", num_cores=NC, num_subcores=NS
    ),
    compiler_params=pltpu.CompilerParams(needs_layout_passes=False),
    scratch_types=[
        pltpu.VMEM((N,), jnp.float32),
        pltpu.VMEM((N,), jnp.float32),
        pltpu.VMEM((N,), jnp.float32),
        pltpu.VMEM((N,), jnp.int32),
        pltpu.VMEM((SPAD,), jnp.int32),
        pltpu.VMEM((16 * K,), jnp.int32),
        pltpu.VMEM((16 * K,), jnp.float32),
    ],
)
def _sc_topk(xh, yh, zh, bh, sh, col_h, d2_h, xv, yv, zv, bv, sv, colb, d2b):
    wid = lax.axis_index("s") * NC + lax.axis_index("c")
    pltpu.sync_copy(xh, xv)
    pltpu.sync_copy(yh, yv)
    pltpu.sync_copy(zh, zv)
    pltpu.sync_copy(bh, bv)
    pltpu.sync_copy(sh, sv)

    lo = (wid * NBLK) // NW
    hi = ((wid + 1) * NBLK) // NW
    lane = lax.iota(jnp.int32, 16)
    inf = jnp.float32(jnp.inf)
    intmin = jnp.int32(-2147483648)

    stab = [sv[pl.ds(i * 16, 16)] for i in range(SPAD // 16)]

    def starts_at(g):
        best = jnp.full((16,), intmin)
        for i, vec in enumerate(stab):
            best = jnp.maximum(best, jnp.where(lane + i * 16 == g, vec, intmin))
        return jnp.max(best)

    def qblock(qb, carry):
        base = qb * 16
        qxv = xv[pl.ds(base, 16)]
        qyv = yv[pl.ds(base, 16)]
        qzv = zv[pl.ds(base, 16)]
        gvec = bv[pl.ds(base, 16)]

        for grp in range(16 // GQ):
            l0 = grp * GQ
            glo = jnp.max(jnp.where(lane == l0, gvec, intmin))
            ghi = jnp.max(jnp.where(lane == l0 + GQ - 1, gvec, intmin))
            cb_lo = starts_at(glo) >> 4
            cb_hi = (starts_at(ghi + 1) + 15) >> 4

            qx = []
            qy = []
            qz = []
            gq = []
            W = []
            for j in range(GQ):
                lm = lane == l0 + j
                qx.append(jnp.full((16,), jnp.max(jnp.where(lm, qxv, -inf))))
                qy.append(jnp.full((16,), jnp.max(jnp.where(lm, qyv, -inf))))
                qz.append(jnp.full((16,), jnp.max(jnp.where(lm, qzv, -inf))))
                gq.append(jnp.max(jnp.where(lm, gvec, intmin)))
                qidx = base + l0 + j
                W.extend([
                    jnp.full((16,), inf),
                    jnp.full((16,), jnp.int32(qidx)),
                    jnp.full((16,), inf),
                    jnp.full((16,), jnp.int32(qidx)),
                ])

            def cand(cb, Wc):
                cbase = cb * 16
                cx = xv[pl.ds(cbase, 16)]
                cy = yv[pl.ds(cbase, 16)]
                cz = zv[pl.ds(cbase, 16)]
                cg = bv[pl.ds(cbase, 16)]
                cidx = cbase + lane
                out = []
                for j in range(GQ):
                    W0k, W0v, W1k, W1v = Wc[4 * j:4 * j + 4]
                    dx = qx[j] - cx
                    dy = qy[j] - cy
                    dz = qz[j] - cz
                    d2 = (dx * dx + dy * dy) + dz * dz
                    qidx = base + l0 + j
                    bad = (
                        (cg != gq[j])
                        | (cidx == qidx)
                        | (d2 > jnp.float32(CUTOFF2))
                    )
                    skey = jnp.where(bad, inf, d2)
                    out.extend(_merge_topk(W0k, W0v, W1k, W1v, skey, cidx))
                return tuple(out)

            W = lax.fori_loop(cb_lo, cb_hi, cand, tuple(W))
            for j in range(GQ):
                W0k, W0v, W1k, W1v = W[4 * j:4 * j + 4]
                qidx = base + l0 + j
                pad0 = W0k == inf
                pad1 = W1k == inf
                o = (l0 + j) * K
                colb[pl.ds(o, 16)] = jnp.where(pad0, jnp.int32(qidx), W0v)
                colb[pl.ds(o + 16, 16)] = jnp.where(pad1, jnp.int32(qidx), W1v)
                d2b[pl.ds(o, 16)] = jnp.where(pad0, jnp.float32(0.0), W0k)
                d2b[pl.ds(o + 16, 16)] = jnp.where(pad1, jnp.float32(0.0), W1k)

        pltpu.sync_copy(colb, col_h.at[pl.ds(qb * (16 * K), 16 * K)])
        pltpu.sync_copy(d2b, d2_h.at[pl.ds(qb * (16 * K), 16 * K)])
        return carry

    lax.fori_loop(lo, hi, qblock, 0)


def _fin_body(d2_ref, row_ref, w_ref):
    d2 = d2_ref[...]
    lin = (
        lax.broadcasted_iota(jnp.int32, (ROWS2D, 128), 0) * 128
        + lax.broadcasted_iota(jnp.int32, (ROWS2D, 128), 1)
    )
    row_ref[...] = lin >> 5
    w_ref[...] = jnp.where(d2 > 0, jnp.sqrt(d2), jnp.float32(0.0))


_finalize = pl.pallas_call(
    _fin_body,
    out_shape=(
        jax.ShapeDtypeStruct((ROWS2D, 128), jnp.int32),
        jax.ShapeDtypeStruct((ROWS2D, 128), jnp.float32),
    ),
)


def kernel(pos, batch):
    pos = pos.astype(jnp.float32)
    b = batch.astype(jnp.int32)
    starts = jnp.searchsorted(
        b, jnp.arange(NGRAPH + 1, dtype=jnp.int32)
    ).astype(jnp.int32)
    starts = jnp.pad(starts, (0, SPAD - (NGRAPH + 1)), constant_values=N)

    col, d2 = _sc_topk(pos[:, 0], pos[:, 1], pos[:, 2], b, starts)
    row2d, w2d = _finalize(d2.reshape(ROWS2D, 128))

    edge_index = jnp.stack([row2d.reshape(-1), col], axis=0)
    return edge_index, w2d.reshape(-1)


# speedup vs baseline: 305.4283x; 1.4011x over previous
import functools

import jax
import jax.numpy as jnp
from jax import lax
from jax.experimental import pallas as pl
from jax.experimental.pallas import tpu as pltpu
from jax.experimental.pallas import tpu_sc as plsc

N = 10000
K = 32
CUTOFF2 = 100.0
NGRAPH = 64
NBLK = N // 16
NC, NS, L = 2, 16, 16
NW = NC * NS
SPAD = 80
ROWS2D = (N * K) // 128
GQ = 4


def _merge_topk(W0k, W0v, W1k, W1v, sk, sv):
    sk, sv = plsc.sort_key_val(sk, sv)
    rsk = jnp.flip(sk, 0)
    rsv = jnp.flip(sv, 0)
    m = W1k <= rsk
    lk = jnp.where(m, W1k, rsk)
    lv = jnp.where(m, W1v, rsv)
    lk, lv = plsc.sort_key_val(lk, lv)
    rlk = jnp.flip(lk, 0)
    rlv = jnp.flip(lv, 0)
    m2 = W0k <= rlk
    mk = jnp.where(m2, W0k, rlk)
    mv = jnp.where(m2, W0v, rlv)
    xk = jnp.where(m2, rlk, W0k)
    xv = jnp.where(m2, rlv, W0v)
    W0k, W0v = plsc.sort_key_val(mk, mv)
    W1k, W1v = plsc.sort_key_val(xk, xv)
    return W0k, W0v, W1k, W1v


@functools.partial(
    pl.kernel,
    out_type=(
        jax.ShapeDtypeStruct((N * K,), jnp.int32),
        jax.ShapeDtypeStruct((N * K,), jnp.float32),
    ),
    mesh=plsc.VectorSubcoreMesh(
        core_axis_name="c", subcore_axis_name="s", num_cores=NC, num_subcores=NS
    ),
    compiler_params=pltpu.CompilerParams(needs_layout_passes=False),
    scratch_types=[
        pltpu.VMEM((N,), jnp.float32),
        pltpu.VMEM((N,), jnp.float32),
        pltpu.VMEM((N,), jnp.float32),
        pltpu.VMEM((N,), jnp.int32),
        pltpu.VMEM((SPAD,), jnp.int32),
        pltpu.VMEM((16 * K,), jnp.int32),
        pltpu.VMEM((16 * K,), jnp.float32),
    ],
)
def _sc_topk(xh, yh, zh, bh, sh, col_h, d2_h, xv, yv, zv, bv, sv, colb, d2b):
    wid = lax.axis_index("s") * NC + lax.axis_index("c")
    pltpu.sync_copy(xh, xv)
    pltpu.sync_copy(yh, yv)
    pltpu.sync_copy(zh, zv)
    pltpu.sync_copy(bh, bv)
    pltpu.sync_copy(sh, sv)

    lo = (wid * NBLK) // NW
    hi = ((wid + 1) * NBLK) // NW
    lane = lax.iota(jnp.int32, 16)
    inf = jnp.float32(jnp.inf)
    intmin = jnp.int32(-2147483648)

    stab = [sv[pl.ds(i * 16, 16)] for i in range(SPAD // 16)]

    def starts_at(g):
        best = jnp.full((16,), intmin)
        for i, vec in enumerate(stab):
            best = jnp.maximum(best, jnp.where(lane + i * 16 == g, vec, intmin))
        return jnp.max(best)

    def qblock(qb, carry):
        base = qb * 16
        qxv = xv[pl.ds(base, 16)]
        qyv = yv[pl.ds(base, 16)]
        qzv = zv[pl.ds(base, 16)]
        gvec = bv[pl.ds(base, 16)]

        for grp in range(16 // GQ):
            l0 = grp * GQ
            glo = jnp.max(jnp.where(lane == l0, gvec, intmin))
            ghi = jnp.max(jnp.where(lane == l0 + GQ - 1, gvec, intmin))
            cb_lo = starts_at(glo) >> 4
            cb_hi = (starts_at(ghi + 1) + 15) >> 4

            qx = []
            qy = []
            qz = []
            gq = []
            W = []
            for j in range(GQ):
                lm = lane == l0 + j
                qx.append(jnp.full((16,), jnp.max(jnp.where(lm, qxv, -inf))))
                qy.append(jnp.full((16,), jnp.max(jnp.where(lm, qyv, -inf))))
                qz.append(jnp.full((16,), jnp.max(jnp.where(lm, qzv, -inf))))
                gq.append(jnp.max(jnp.where(lm, gvec, intmin)))
                qidx = base + l0 + j
                W.extend([
                    jnp.full((16,), inf),
                    jnp.full((16,), jnp.int32(qidx)),
                    jnp.full((16,), inf),
                    jnp.full((16,), jnp.int32(qidx)),
                ])

            def cand(cb, Wc):
                cbase = cb * 16
                cx = xv[pl.ds(cbase, 16)]
                cy = yv[pl.ds(cbase, 16)]
                cz = zv[pl.ds(cbase, 16)]
                cg = bv[pl.ds(cbase, 16)]
                cidx = cbase + lane
                out = []
                for j in range(GQ):
                    W0k, W0v, W1k, W1v = Wc[4 * j:4 * j + 4]
                    dx = qx[j] - cx
                    dy = qy[j] - cy
                    dz = qz[j] - cz
                    d2 = (dx * dx + dy * dy) + dz * dz
                    qidx = base + l0 + j
                    bad = (
                        (cg != gq[j])
                        | (cidx == qidx)
                        | (d2 > jnp.float32(CUTOFF2))
                    )
                    skey = jnp.where(bad, inf, d2)
                    out.extend(_merge_topk(W0k, W0v, W1k, W1v, skey, cidx))
                return tuple(out)

            W = lax.fori_loop(cb_lo, cb_hi, cand, tuple(W))
            for j in range(GQ):
                W0k, W0v, W1k, W1v = W[4 * j:4 * j + 4]
                qidx = base + l0 + j
                pad0 = W0k == inf
                pad1 = W1k == inf
                o = (l0 + j) * K
                colb[pl.ds(o, 16)] = jnp.where(pad0, jnp.int32(qidx), W0v)
                colb[pl.ds(o + 16, 16)] = jnp.where(pad1, jnp.int32(qidx), W1v)
                d2b[pl.ds(o, 16)] = jnp.where(pad0, jnp.float32(0.0), W0k)
                d2b[pl.ds(o + 16, 16)] = jnp.where(pad1, jnp.float32(0.0), W1k)

        pltpu.sync_copy(colb, col_h.at[pl.ds(qb * (16 * K), 16 * K)])
        pltpu.sync_copy(d2b, d2_h.at[pl.ds(qb * (16 * K), 16 * K)])
        return carry

    lax.fori_loop(lo, hi, qblock, 0)


def _fin_body(col_ref, d2_ref, ei_ref, w_ref):
    d2 = d2_ref[...]
    lin = (
        lax.broadcasted_iota(jnp.int32, (ROWS2D, 128), 0) * 128
        + lax.broadcasted_iota(jnp.int32, (ROWS2D, 128), 1)
    )
    ei_ref[:ROWS2D, :] = lin >> 5
    ei_ref[ROWS2D:, :] = col_ref[...]
    w_ref[...] = jnp.where(d2 > 0, jnp.sqrt(d2), jnp.float32(0.0))


_finalize = pl.pallas_call(
    _fin_body,
    out_shape=(
        jax.ShapeDtypeStruct((2 * ROWS2D, 128), jnp.int32),
        jax.ShapeDtypeStruct((ROWS2D, 128), jnp.float32),
    ),
)


def _seg_starts(b):
    g = jnp.arange(NGRAPH + 1, dtype=jnp.int32)
    starts = jnp.sum(
        (b[None, :] < g[:, None]).astype(jnp.int32), axis=1, dtype=jnp.int32
    )
    return jnp.pad(starts, (0, SPAD - (NGRAPH + 1)), constant_values=N)


def kernel(pos, batch):
    pos = pos.astype(jnp.float32)
    b = batch.astype(jnp.int32)
    col, d2 = _sc_topk(pos[:, 0], pos[:, 1], pos[:, 2], b, _seg_starts(b))
    ei2d, w2d = _finalize(col.reshape(ROWS2D, 128), d2.reshape(ROWS2D, 128))
    return ei2d.reshape(2, N * K), w2d.reshape(-1)


# speedup vs baseline: 316.9604x; 1.0378x over previous
import functools

import jax
import jax.numpy as jnp
from jax import lax
from jax.experimental import pallas as pl
from jax.experimental.pallas import tpu as pltpu
from jax.experimental.pallas import tpu_sc as plsc

N = 10000
K = 32
CUTOFF2 = 100.0
NGRAPH = 64
NBLK = N // 16
NC, NS, L = 2, 16, 16
NW = NC * NS
SPAD = 80
ROWS2D = (N * K) // 128
GQ = 4


def _merge_topk(W0k, W0v, W1k, W1v, sk, sv):
    sk, sv = plsc.sort_key_val(sk, sv)
    rsk = jnp.flip(sk, 0)
    rsv = jnp.flip(sv, 0)
    m = W1k <= rsk
    lk = jnp.where(m, W1k, rsk)
    lv = jnp.where(m, W1v, rsv)
    lk, lv = plsc.sort_key_val(lk, lv)
    rlk = jnp.flip(lk, 0)
    rlv = jnp.flip(lv, 0)
    m2 = W0k <= rlk
    mk = jnp.where(m2, W0k, rlk)
    mv = jnp.where(m2, W0v, rlv)
    xk = jnp.where(m2, rlk, W0k)
    xv = jnp.where(m2, rlv, W0v)
    W0k, W0v = plsc.sort_key_val(mk, mv)
    W1k, W1v = plsc.sort_key_val(xk, xv)
    return W0k, W0v, W1k, W1v


@functools.partial(
    pl.kernel,
    out_type=(
        jax.ShapeDtypeStruct((N * K,), jnp.int32),
        jax.ShapeDtypeStruct((N * K,), jnp.float32),
    ),
    mesh=plsc.VectorSubcoreMesh(
        core_axis_name="c", subcore_axis_name="s", num_cores=NC, num_subcores=NS
    ),
    compiler_params=pltpu.CompilerParams(needs_layout_passes=False),
    scratch_types=[
        pltpu.VMEM((N,), jnp.float32),
        pltpu.VMEM((N,), jnp.float32),
        pltpu.VMEM((N,), jnp.float32),
        pltpu.VMEM((N,), jnp.int32),
        pltpu.VMEM((SPAD,), jnp.int32),
        pltpu.VMEM((16 * K,), jnp.int32),
        pltpu.VMEM((16 * K,), jnp.float32),
        pltpu.SemaphoreType.DMA,
    ],
)
def _sc_topk(xh, yh, zh, bh, sh, col_h, d2_h, xv, yv, zv, bv, sv, colb, d2b,
             sem):
    wid = lax.axis_index("s") * NC + lax.axis_index("c")
    pltpu.sync_copy(xh, xv)
    pltpu.sync_copy(yh, yv)
    pltpu.sync_copy(zh, zv)
    pltpu.sync_copy(bh, bv)
    pltpu.sync_copy(sh, sv)

    lo = (wid * NBLK) // NW
    hi = ((wid + 1) * NBLK) // NW
    lane = lax.iota(jnp.int32, 16)
    inf = jnp.float32(jnp.inf)
    intmin = jnp.int32(-2147483648)

    stab = [sv[pl.ds(i * 16, 16)] for i in range(SPAD // 16)]

    def starts_at(g):
        best = jnp.full((16,), intmin)
        for i, vec in enumerate(stab):
            best = jnp.maximum(best, jnp.where(lane + i * 16 == g, vec, intmin))
        return jnp.max(best)

    def qblock(qb, carry):
        base = qb * 16
        qxv = xv[pl.ds(base, 16)]
        qyv = yv[pl.ds(base, 16)]
        qzv = zv[pl.ds(base, 16)]
        gvec = bv[pl.ds(base, 16)]

        for grp in range(16 // GQ):
            l0 = grp * GQ
            glo = jnp.max(jnp.where(lane == l0, gvec, intmin))
            ghi = jnp.max(jnp.where(lane == l0 + GQ - 1, gvec, intmin))
            cb_lo = starts_at(glo) >> 4
            cb_hi = (starts_at(ghi + 1) + 15) >> 4

            qx = []
            qy = []
            qz = []
            gq = []
            W = []
            for j in range(GQ):
                lm = lane == l0 + j
                qx.append(jnp.full((16,), jnp.max(jnp.where(lm, qxv, -inf))))
                qy.append(jnp.full((16,), jnp.max(jnp.where(lm, qyv, -inf))))
                qz.append(jnp.full((16,), jnp.max(jnp.where(lm, qzv, -inf))))
                gq.append(jnp.max(jnp.where(lm, gvec, intmin)))
                qidx = base + l0 + j
                W.extend([
                    jnp.full((16,), inf),
                    jnp.full((16,), jnp.int32(qidx)),
                    jnp.full((16,), inf),
                    jnp.full((16,), jnp.int32(qidx)),
                ])

            def cand(cb, Wc):
                cbase = cb * 16
                cx = xv[pl.ds(cbase, 16)]
                cy = yv[pl.ds(cbase, 16)]
                cz = zv[pl.ds(cbase, 16)]
                cg = bv[pl.ds(cbase, 16)]
                cidx = cbase + lane
                out = []
                for j in range(GQ):
                    W0k, W0v, W1k, W1v = Wc[4 * j:4 * j + 4]
                    dx = qx[j] - cx
                    dy = qy[j] - cy
                    dz = qz[j] - cz
                    d2 = (dx * dx + dy * dy) + dz * dz
                    qidx = base + l0 + j
                    bad = (
                        (cg != gq[j])
                        | (cidx == qidx)
                        | (d2 > jnp.float32(CUTOFF2))
                    )
                    skey = jnp.where(bad, inf, d2)
                    out.extend(_merge_topk(W0k, W0v, W1k, W1v, skey, cidx))
                return tuple(out)

            W = lax.fori_loop(cb_lo, cb_hi, cand, tuple(W))

            if grp == 0:
                @pl.when(qb > lo)
                def _():
                    pltpu.make_async_copy(
                        colb, col_h.at[pl.ds(qb * (16 * K), 16 * K)], sem
                    ).wait()
                    pltpu.make_async_copy(
                        d2b, d2_h.at[pl.ds(qb * (16 * K), 16 * K)], sem
                    ).wait()

            for j in range(GQ):
                W0k, W0v, W1k, W1v = W[4 * j:4 * j + 4]
                qidx = base + l0 + j
                pad0 = W0k == inf
                pad1 = W1k == inf
                o = (l0 + j) * K
                colb[pl.ds(o, 16)] = jnp.where(pad0, jnp.int32(qidx), W0v)
                colb[pl.ds(o + 16, 16)] = jnp.where(pad1, jnp.int32(qidx), W1v)
                d2b[pl.ds(o, 16)] = jnp.where(pad0, jnp.float32(0.0), W0k)
                d2b[pl.ds(o + 16, 16)] = jnp.where(pad1, jnp.float32(0.0), W1k)

        pltpu.async_copy(colb, col_h.at[pl.ds(qb * (16 * K), 16 * K)], sem)
        pltpu.async_copy(d2b, d2_h.at[pl.ds(qb * (16 * K), 16 * K)], sem)
        return carry

    lax.fori_loop(lo, hi, qblock, 0)

    @pl.when(hi > lo)
    def _():
        pltpu.make_async_copy(
            colb, col_h.at[pl.ds((hi - 1) * (16 * K), 16 * K)], sem
        ).wait()
        pltpu.make_async_copy(
            d2b, d2_h.at[pl.ds((hi - 1) * (16 * K), 16 * K)], sem
        ).wait()


def _fin_body(col_ref, d2_ref, ei_ref, w_ref):
    d2 = d2_ref[...]
    lin = (
        lax.broadcasted_iota(jnp.int32, (ROWS2D, 128), 0) * 128
        + lax.broadcasted_iota(jnp.int32, (ROWS2D, 128), 1)
    )
    ei_ref[:ROWS2D, :] = lin >> 5
    ei_ref[ROWS2D:, :] = col_ref[...]
    w_ref[...] = jnp.where(d2 > 0, jnp.sqrt(d2), jnp.float32(0.0))


_finalize = pl.pallas_call(
    _fin_body,
    out_shape=(
        jax.ShapeDtypeStruct((2 * ROWS2D, 128), jnp.int32),
        jax.ShapeDtypeStruct((ROWS2D, 128), jnp.float32),
    ),
)


def _seg_starts(b):
    g = jnp.arange(NGRAPH + 1, dtype=jnp.int32)
    starts = jnp.sum(
        (b[None, :] < g[:, None]).astype(jnp.int32), axis=1, dtype=jnp.int32
    )
    return jnp.pad(starts, (0, SPAD - (NGRAPH + 1)), constant_values=N)


def kernel(pos, batch):
    pos = pos.astype(jnp.float32)
    b = batch.astype(jnp.int32)
    col, d2 = _sc_topk(pos[:, 0], pos[:, 1], pos[:, 2], b, _seg_starts(b))
    ei2d, w2d = _finalize(col.reshape(ROWS2D, 128), d2.reshape(ROWS2D, 128))
    return ei2d.reshape(2, N * K), w2d.reshape(-1)
